# Initial kernel scaffold; baseline (speedup 1.0000x reference)
import jax
import jax.numpy as jnp
from jax.experimental import pallas as pl


def kernel(x, Ws, bs, W1, b1, W2, b2):
    raise NotImplementedError("write your pallas kernel here")


# speedup vs baseline: 2.9027x; 2.9027x over previous
import jax
import jax.numpy as jnp
from jax import lax
from jax.experimental import pallas as pl
from jax.experimental.pallas import tpu as pltpu
from jax.experimental.pallas import tpu_sc as plsc

_B, _S, _D, _F, _E = 2, 2048, 1024, 4096, 8
_T = _B * _S
_M = 256
_NT = 24
_SB = _NT * _M
_NTP = 32

_NW = 32
_TPW = _T // _NW
_CW = 64
_CH = _TPW // _CW


def _router_body(x_ref, ws_ref, bs_ref, pos_ref, p16_ref, te_ref):
    xf = x_ref[...]
    logits = jnp.dot(xf, ws_ref[...],
                     preferred_element_type=jnp.float32) + bs_ref[...]
    m = jnp.max(logits, axis=1, keepdims=True)
    ex = jnp.exp(logits - m)
    ssum = jnp.sum(ex, axis=1, keepdims=True)
    exmax = jnp.max(ex, axis=1, keepdims=True)
    pmax = exmax / ssum

    eidx = lax.broadcasted_iota(jnp.int32, (_T, _E), 1)
    route = jnp.min(jnp.where(ex == exmax, eidx, _E), axis=1, keepdims=True)
    oh = (eidx == route).astype(jnp.float32)

    csum = oh
    k = 1
    while k < _T:
        csum = csum + jnp.concatenate(
            [jnp.zeros((k, _E), jnp.float32), csum[:-k, :]], axis=0)
        k *= 2
    counts = lax.slice(csum, (_T - 1, 0), (_T, _E))
    ntiles = jnp.ceil(counts * (1.0 / _M))
    tcum = ntiles
    for k in (1, 2, 4):
        tcum = tcum + jnp.concatenate(
            [jnp.zeros((1, k), jnp.float32), tcum[:, :-k]], axis=1)
    tstart = tcum - ntiles

    rank = jnp.sum(jnp.where(oh > 0, csum, 0.0), axis=1, keepdims=True)
    base = jnp.sum(jnp.where(oh > 0,
                             jnp.broadcast_to(tstart * _M, (_T, _E)),
                             0.0), axis=1, keepdims=True)
    pos_ref[...] = (base + rank - 1.0).astype(jnp.int32)
    p16_ref[...] = jnp.broadcast_to(pmax, (_T, 128))

    ti = lax.broadcasted_iota(jnp.int32, (1, _NTP), 1).astype(jnp.float32)
    te = jnp.zeros((1, _NTP), jnp.float32)
    for e in range(_E):
        te = te + (ti >= lax.slice(tcum, (0, e), (1, e + 1))).astype(
            jnp.float32)
    te = jnp.minimum(te, float(_E - 1))
    total = lax.slice(tcum, (0, _E - 1), (1, _E))
    valid = (ti < total).astype(jnp.float32)
    te_ref[...] = jnp.concatenate([te, valid], axis=0).astype(jnp.int32)


def _router(xf, Ws, bs2):
    return pl.pallas_call(
        _router_body,
        out_shape=(
            jax.ShapeDtypeStruct((_T, 1), jnp.int32),
            jax.ShapeDtypeStruct((_T, 128), jnp.float32),
            jax.ShapeDtypeStruct((2, _NTP), jnp.int32),
        ),
    )(xf, Ws, bs2)


def _sc_mesh():
    return plsc.VectorSubcoreMesh(core_axis_name="c", subcore_axis_name="s")


def _dispatch_body(xf_hbm, p_hbm, pos_hbm, xs_hbm, ps_hbm,
                   idx_v, xbuf, pbuf, sem):
    w = lax.axis_index("c") * 16 + lax.axis_index("s")
    base = w * _TPW
    pltpu.sync_copy(pos_hbm.at[w], idx_v)
    for j in range(_CH):
        pltpu.sync_copy(xf_hbm.at[pl.ds(base + j * _CW, _CW)], xbuf)
        pltpu.async_copy(xbuf, xs_hbm.at[idx_v.at[j]], sem).wait()
        pltpu.sync_copy(p_hbm.at[pl.ds(base + j * _CW, _CW)], pbuf)
        pltpu.async_copy(pbuf, ps_hbm.at[idx_v.at[j]], sem).wait()


def _dispatch(xf, p16, pos3):
    return pl.kernel(
        _dispatch_body,
        out_type=(
            jax.ShapeDtypeStruct((_SB, _D), jnp.float32),
            jax.ShapeDtypeStruct((_SB, 128), jnp.float32),
        ),
        mesh=_sc_mesh(),
        scratch_types=[
            pltpu.VMEM((_CH, _CW), jnp.int32),
            pltpu.VMEM((_CW, _D), jnp.float32),
            pltpu.VMEM((_CW, 128), jnp.float32),
            pltpu.SemaphoreType.DMA,
        ],
    )(xf, p16, pos3)


def _ffn1_body(sp_ref, xs_ref, w1_ref, b1_ref, h_ref):
    i = pl.program_id(0)

    @pl.when(sp_ref[_NTP + i] == 1)
    def _():
        h = jnp.dot(xs_ref[...], w1_ref[0],
                    preferred_element_type=jnp.float32)
        h_ref[...] = jnp.maximum(h + b1_ref[0], 0.0)


def _ffn2_body(sp_ref, h_ref, w2_ref, b2_ref, p_ref, o_ref):
    i = pl.program_id(0)

    @pl.when(sp_ref[_NTP + i] == 1)
    def _():
        o = jnp.dot(h_ref[...], w2_ref[0],
                    preferred_element_type=jnp.float32) + b2_ref[0]
        o_ref[...] = o * p_ref[:, 0:1]


def _ffn1(sp, xs, W1, b1r):
    grid_spec = pltpu.PrefetchScalarGridSpec(
        num_scalar_prefetch=1,
        grid=(_NT,),
        in_specs=[
            pl.BlockSpec((_M, _D), lambda i, sp: (i, 0)),
            pl.BlockSpec((1, _D, _F), lambda i, sp: (sp[i], 0, 0)),
            pl.BlockSpec((1, 1, _F), lambda i, sp: (sp[i], 0, 0)),
        ],
        out_specs=pl.BlockSpec((_M, _F), lambda i, sp: (i, 0)),
    )
    return pl.pallas_call(
        _ffn1_body,
        grid_spec=grid_spec,
        out_shape=jax.ShapeDtypeStruct((_SB, _F), jnp.float32),
    )(sp, xs, W1, b1r)


def _ffn2(sp, h, W2, b2r, ps):
    grid_spec = pltpu.PrefetchScalarGridSpec(
        num_scalar_prefetch=1,
        grid=(_NT,),
        in_specs=[
            pl.BlockSpec((_M, _F), lambda i, sp: (i, 0)),
            pl.BlockSpec((1, _F, _D), lambda i, sp: (sp[i], 0, 0)),
            pl.BlockSpec((1, 1, _D), lambda i, sp: (sp[i], 0, 0)),
            pl.BlockSpec((_M, 128), lambda i, sp: (i, 0)),
        ],
        out_specs=pl.BlockSpec((_M, _D), lambda i, sp: (i, 0)),
    )
    return pl.pallas_call(
        _ffn2_body,
        grid_spec=grid_spec,
        out_shape=jax.ShapeDtypeStruct((_SB, _D), jnp.float32),
    )(sp, h, W2, b2r, ps)


def _combine_body(os_hbm, pos_hbm, out_hbm, idx_v, buf, sem):
    w = lax.axis_index("c") * 16 + lax.axis_index("s")
    base = w * _TPW
    pltpu.sync_copy(pos_hbm.at[w], idx_v)
    for j in range(_CH):
        pltpu.async_copy(os_hbm.at[idx_v.at[j]], buf, sem).wait()
        pltpu.sync_copy(buf, out_hbm.at[pl.ds(base + j * _CW, _CW)])


def _combine(os_, pos3):
    return pl.kernel(
        _combine_body,
        out_type=jax.ShapeDtypeStruct((_T, _D), jnp.float32),
        mesh=_sc_mesh(),
        scratch_types=[
            pltpu.VMEM((_CH, _CW), jnp.int32),
            pltpu.VMEM((_CW, _D), jnp.float32),
            pltpu.SemaphoreType.DMA,
        ],
    )(os_, pos3)


def kernel(x, Ws, bs, W1, b1, W2, b2):
    b, s, d = x.shape
    xf = x.reshape(-1, d)
    pos, p16, tev = _router(xf, Ws, bs.reshape(1, _E))
    sp = tev.reshape(-1)
    pos3 = pos.reshape(_NW, _CH, _CW)
    xs, ps = _dispatch(xf, p16, pos3)
    h = _ffn1(sp, xs, W1, b1.reshape(_E, 1, _F))
    os_ = _ffn2(sp, h, W2, b2.reshape(_E, 1, _D), ps)
    out = _combine(os_, pos3)
    return out.reshape(b, s, d)


# speedup vs baseline: 3.1389x; 1.0814x over previous
import jax
import jax.numpy as jnp
from jax import lax
from jax.experimental import pallas as pl
from jax.experimental.pallas import tpu as pltpu
from jax.experimental.pallas import tpu_sc as plsc

_B, _S, _D, _F, _E = 2, 2048, 1024, 4096, 8
_T = _B * _S
_M = 256
_NT = 24
_SB = _NT * _M
_NTP = 32

_NW = 32
_TPW = _T // _NW
_CW = 64
_CH = _TPW // _CW


def _router_body(x_ref, ws_ref, bs_ref, pos_ref, p16_ref, te_ref):
    xf = x_ref[...]
    logits = jnp.dot(xf, ws_ref[...],
                     preferred_element_type=jnp.float32) + bs_ref[...]
    m = jnp.max(logits, axis=1, keepdims=True)
    ex = jnp.exp(logits - m)
    ssum = jnp.sum(ex, axis=1, keepdims=True)
    exmax = jnp.max(ex, axis=1, keepdims=True)
    pmax = exmax / ssum

    eidx = lax.broadcasted_iota(jnp.int32, (_T, _E), 1)
    route = jnp.min(jnp.where(ex == exmax, eidx, _E), axis=1, keepdims=True)
    oh = (eidx == route).astype(jnp.float32)

    csum = oh
    k = 1
    while k < _T:
        csum = csum + jnp.concatenate(
            [jnp.zeros((k, _E), jnp.float32), csum[:-k, :]], axis=0)
        k *= 2
    counts = lax.slice(csum, (_T - 1, 0), (_T, _E))
    ntiles = jnp.ceil(counts * (1.0 / _M))
    tcum = ntiles
    for k in (1, 2, 4):
        tcum = tcum + jnp.concatenate(
            [jnp.zeros((1, k), jnp.float32), tcum[:, :-k]], axis=1)
    tstart = tcum - ntiles

    rank = jnp.sum(jnp.where(oh > 0, csum, 0.0), axis=1, keepdims=True)
    base = jnp.sum(jnp.where(oh > 0,
                             jnp.broadcast_to(tstart * _M, (_T, _E)),
                             0.0), axis=1, keepdims=True)
    pos_ref[...] = (base + rank - 1.0).astype(jnp.int32)
    p16_ref[...] = jnp.broadcast_to(pmax, (_T, 128))

    ti = lax.broadcasted_iota(jnp.int32, (1, _NTP), 1).astype(jnp.float32)
    te = jnp.zeros((1, _NTP), jnp.float32)
    for e in range(_E):
        te = te + (ti >= lax.slice(tcum, (0, e), (1, e + 1))).astype(
            jnp.float32)
    te = jnp.minimum(te, float(_E - 1))
    total = lax.slice(tcum, (0, _E - 1), (1, _E))
    valid = (ti < total).astype(jnp.float32)
    te_ref[...] = jnp.concatenate([te, valid], axis=0).astype(jnp.int32)


def _router(xf, Ws, bs2):
    return pl.pallas_call(
        _router_body,
        out_shape=(
            jax.ShapeDtypeStruct((_T, 1), jnp.int32),
            jax.ShapeDtypeStruct((_T, 128), jnp.float32),
            jax.ShapeDtypeStruct((2, _NTP), jnp.int32),
        ),
    )(xf, Ws, bs2)


def _sc_mesh():
    return plsc.VectorSubcoreMesh(core_axis_name="c", subcore_axis_name="s")


def _dispatch_body(xf_hbm, p_hbm, pos_hbm, xs_hbm, ps_hbm,
                   idx_v, xbuf, pbuf, sem):
    w = lax.axis_index("c") * 16 + lax.axis_index("s")
    base = w * _TPW
    pltpu.sync_copy(pos_hbm.at[w], idx_v)
    for j in range(_CH):
        pltpu.sync_copy(xf_hbm.at[pl.ds(base + j * _CW, _CW)], xbuf)
        pltpu.async_copy(xbuf, xs_hbm.at[idx_v.at[j]], sem).wait()
        pltpu.sync_copy(p_hbm.at[pl.ds(base + j * _CW, _CW)], pbuf)
        pltpu.async_copy(pbuf, ps_hbm.at[idx_v.at[j]], sem).wait()


def _dispatch(xf, p16, pos3):
    return pl.kernel(
        _dispatch_body,
        out_type=(
            jax.ShapeDtypeStruct((_SB, _D), jnp.float32),
            jax.ShapeDtypeStruct((_SB, 128), jnp.float32),
        ),
        mesh=_sc_mesh(),
        scratch_types=[
            pltpu.VMEM((_CH, _CW), jnp.int32),
            pltpu.VMEM((_CW, _D), jnp.float32),
            pltpu.VMEM((_CW, 128), jnp.float32),
            pltpu.SemaphoreType.DMA,
        ],
    )(xf, p16, pos3)


def _ffn1_body(sp_ref, xs_ref, w1_ref, b1_ref, h_ref):
    i = pl.program_id(0)

    @pl.when(sp_ref[_NTP + i] == 1)
    def _():
        h = jnp.dot(xs_ref[...].astype(jnp.bfloat16),
                    w1_ref[0].astype(jnp.bfloat16),
                    preferred_element_type=jnp.float32)
        h_ref[...] = jnp.maximum(h + b1_ref[0], 0.0).astype(jnp.bfloat16)


def _ffn2_body(sp_ref, h_ref, w2_ref, b2_ref, p_ref, o_ref):
    i = pl.program_id(0)

    @pl.when(sp_ref[_NTP + i] == 1)
    def _():
        o = jnp.dot(h_ref[...], w2_ref[0].astype(jnp.bfloat16),
                    preferred_element_type=jnp.float32) + b2_ref[0]
        o_ref[...] = o * p_ref[:, 0:1]


def _ffn1(sp, xs, W1, b1r):
    grid_spec = pltpu.PrefetchScalarGridSpec(
        num_scalar_prefetch=1,
        grid=(_NT,),
        in_specs=[
            pl.BlockSpec((_M, _D), lambda i, sp: (i, 0)),
            pl.BlockSpec((1, _D, _F), lambda i, sp: (sp[i], 0, 0)),
            pl.BlockSpec((1, 1, _F), lambda i, sp: (sp[i], 0, 0)),
        ],
        out_specs=pl.BlockSpec((_M, _F), lambda i, sp: (i, 0)),
    )
    return pl.pallas_call(
        _ffn1_body,
        grid_spec=grid_spec,
        out_shape=jax.ShapeDtypeStruct((_SB, _F), jnp.bfloat16),
    )(sp, xs, W1, b1r)


def _ffn2(sp, h, W2, b2r, ps):
    grid_spec = pltpu.PrefetchScalarGridSpec(
        num_scalar_prefetch=1,
        grid=(_NT,),
        in_specs=[
            pl.BlockSpec((_M, _F), lambda i, sp: (i, 0)),
            pl.BlockSpec((1, _F, _D), lambda i, sp: (sp[i], 0, 0)),
            pl.BlockSpec((1, 1, _D), lambda i, sp: (sp[i], 0, 0)),
            pl.BlockSpec((_M, 128), lambda i, sp: (i, 0)),
        ],
        out_specs=pl.BlockSpec((_M, _D), lambda i, sp: (i, 0)),
    )
    return pl.pallas_call(
        _ffn2_body,
        grid_spec=grid_spec,
        out_shape=jax.ShapeDtypeStruct((_SB, _D), jnp.float32),
    )(sp, h, W2, b2r, ps)


def _combine_body(os_hbm, pos_hbm, out_hbm, idx_v, buf, sem):
    w = lax.axis_index("c") * 16 + lax.axis_index("s")
    base = w * _TPW
    pltpu.sync_copy(pos_hbm.at[w], idx_v)
    for j in range(_CH):
        pltpu.async_copy(os_hbm.at[idx_v.at[j]], buf, sem).wait()
        pltpu.sync_copy(buf, out_hbm.at[pl.ds(base + j * _CW, _CW)])


def _combine(os_, pos3):
    return pl.kernel(
        _combine_body,
        out_type=jax.ShapeDtypeStruct((_T, _D), jnp.float32),
        mesh=_sc_mesh(),
        scratch_types=[
            pltpu.VMEM((_CH, _CW), jnp.int32),
            pltpu.VMEM((_CW, _D), jnp.float32),
            pltpu.SemaphoreType.DMA,
        ],
    )(os_, pos3)


def kernel(x, Ws, bs, W1, b1, W2, b2):
    b, s, d = x.shape
    xf = x.reshape(-1, d)
    pos, p16, tev = _router(xf, Ws, bs.reshape(1, _E))
    sp = tev.reshape(-1)
    pos3 = pos.reshape(_NW, _CH, _CW)
    xs, ps = _dispatch(xf, p16, pos3)
    h = _ffn1(sp, xs, W1, b1.reshape(_E, 1, _F))
    os_ = _ffn2(sp, h, W2, b2.reshape(_E, 1, _D), ps)
    out = _combine(os_, pos3)
    return out.reshape(b, s, d)
